# baseline (speedup 1.0000x reference)
import functools

import jax
import jax.numpy as jnp
from jax import lax
from jax.experimental import pallas as pl
from jax.experimental.pallas import tpu as pltpu
from jax.experimental.pallas import tpu_sc as plsc

BATCH = 16384
D = 128
NC = 2
NS = 16
NW = NC * NS
BPW = BATCH // NW
CHUNK = 128
NCH = BPW // CHUNK


@functools.lru_cache(maxsize=4)
def _make_gather(nrows):
    bpw = nrows // NW
    nch = bpw // CHUNK

    def body(table, idx_hbm, out, idx_v, rows_v, sem):
        wid = lax.axis_index("s") * NC + lax.axis_index("c")
        base = wid * bpw
        pltpu.sync_copy(idx_hbm.at[wid], idx_v)
        cps = [
            pltpu.async_copy(table.at[idx_v.at[c]],
                             rows_v.at[pl.ds(c * CHUNK, CHUNK)], sem)
            for c in range(nch)
        ]
        for cp in cps:
            cp.wait()
        pltpu.sync_copy(rows_v, out.at[pl.ds(base, bpw)])

    mesh = plsc.VectorSubcoreMesh(core_axis_name="c", subcore_axis_name="s")
    return functools.partial(
        pl.kernel,
        mesh=mesh,
        out_type=jax.ShapeDtypeStruct((nrows, D), jnp.float32),
        scratch_types=[
            pltpu.VMEM((nch, CHUNK), jnp.int32),
            pltpu.VMEM((bpw, D), jnp.float32),
            pltpu.SemaphoreType.DMA,
        ],
    )(body)


BLK = 2048


def _mlp_body(xv, w1, b1, w2, b2, ones, out):
    x = xv[...]
    h = lax.dot_general(x, w1[...], (((1,), (1,)), ((), ())),
                        preferred_element_type=jnp.float32)
    h = jnp.maximum(h + b1[...], 0.0)
    y = lax.dot_general(h, w2[...], (((1,), (1,)), ((), ())),
                        preferred_element_type=jnp.float32) + b2[...]
    n2 = lax.dot_general(y * y, ones[...], (((1,), (0,)), ((), ())),
                         preferred_element_type=jnp.float32)
    inv = jnp.where(n2 <= 1e-24, 1e12, lax.rsqrt(n2))
    out[...] = y * inv


def _mlp_half(half, carry, xv, w1, b1, w2, b2, ones):
    nblk = BATCH // 2 // BLK
    vec_spec = pl.BlockSpec((BLK, D), lambda i: (i, 0))
    w_spec = pl.BlockSpec((D, D), lambda i: (0, 0))
    b_spec = pl.BlockSpec((1, D), lambda i: (0, 0))
    out_spec = pl.BlockSpec((BLK, D), lambda i, h=half: (h * nblk + i, 0))
    in_specs = [vec_spec, w_spec, b_spec, w_spec, b_spec, w_spec]
    args = (xv, w1, b1.reshape(1, D), w2, b2.reshape(1, D), ones)
    out_shape = jax.ShapeDtypeStruct((BATCH, D), jnp.float32)
    if carry is None:
        return pl.pallas_call(
            _mlp_body,
            grid=(nblk,),
            in_specs=in_specs,
            out_specs=out_spec,
            out_shape=out_shape,
        )(*args)
    return pl.pallas_call(
        lambda xv, w1, b1, w2, b2, ones, c, o: _mlp_body(
            xv, w1, b1, w2, b2, ones, o),
        grid=(nblk,),
        in_specs=in_specs + [pl.BlockSpec(memory_space=pl.ANY)],
        out_specs=out_spec,
        out_shape=out_shape,
        input_output_aliases={6: 0},
    )(*args, carry)


def kernel(user_ids, item_ids, user_table, item_table,
           W1u, b1u, W2u, b2u, W1i, b1i, W2i, b2i):
    half = BATCH // 2
    uidx = user_ids.astype(jnp.int32).reshape(2, NW, half // NW // CHUNK,
                                              CHUNK)
    iidx = item_ids.astype(jnp.int32).reshape(NW, NCH, CHUNK)
    ghalf = _make_gather(half)
    gfull = _make_gather(BATCH)
    uv0 = ghalf(user_table, uidx[0])
    uv1 = ghalf(user_table, uidx[1])
    v_vecs = gfull(item_table, iidx)
    ones = jnp.ones((D, D), jnp.float32)
    u = _mlp_half(0, None, uv0, W1u, b1u, W2u, b2u, ones)
    u = _mlp_half(1, u, uv1, W1u, b1u, W2u, b2u, ones)
    vec_spec = pl.BlockSpec((BLK, D), lambda i: (i, 0))
    w_spec = pl.BlockSpec((D, D), lambda i: (0, 0))
    b_spec = pl.BlockSpec((1, D), lambda i: (0, 0))
    v = pl.pallas_call(
        _mlp_body,
        grid=(BATCH // BLK,),
        in_specs=[vec_spec, w_spec, b_spec, w_spec, b_spec, w_spec],
        out_specs=vec_spec,
        out_shape=jax.ShapeDtypeStruct((BATCH, D), jnp.float32),
    )(v_vecs, W1i, b1i.reshape(1, D), W2i, b2i.reshape(1, D), ones)
    return (u, v)


# speedup vs baseline: 1.1457x; 1.1457x over previous
import functools

import jax
import jax.numpy as jnp
from jax import lax
from jax.experimental import pallas as pl
from jax.experimental.pallas import tpu as pltpu
from jax.experimental.pallas import tpu_sc as plsc

BATCH = 16384
D = 128
NC = 2
NS = 16
NW = NC * NS
BPW = BATCH // NW
CHUNK = 128
NCH = BPW // CHUNK


def _gather_body(table, idx_hbm, out, idx_v, rows_v, sem):
    wid = lax.axis_index("s") * NC + lax.axis_index("c")
    base = wid * BPW
    pltpu.sync_copy(idx_hbm.at[wid], idx_v)
    cps = [
        pltpu.async_copy(table.at[idx_v.at[c]],
                         rows_v.at[pl.ds(c * CHUNK, CHUNK)], sem)
        for c in range(NCH)
    ]
    for cp in cps:
        cp.wait()
    pltpu.sync_copy(rows_v, out.at[pl.ds(base, BPW)])


@functools.lru_cache(maxsize=1)
def _make_gather():
    mesh = plsc.VectorSubcoreMesh(core_axis_name="c", subcore_axis_name="s")
    return functools.partial(
        pl.kernel,
        mesh=mesh,
        out_type=jax.ShapeDtypeStruct((BATCH, D), jnp.float32),
        scratch_types=[
            pltpu.VMEM((NCH, CHUNK), jnp.int32),
            pltpu.VMEM((BPW, D), jnp.float32),
            pltpu.SemaphoreType.DMA,
        ],
    )(_gather_body)


BLK = 4096


def _mlp_body(xv, w1, b1, w2, b2, ones, out):
    x = xv[...]
    h = lax.dot_general(x, w1[...], (((1,), (1,)), ((), ())),
                        preferred_element_type=jnp.float32)
    h = jnp.maximum(h + b1[...], 0.0)
    y = lax.dot_general(h, w2[...], (((1,), (1,)), ((), ())),
                        preferred_element_type=jnp.float32) + b2[...]
    n2 = lax.dot_general(y * y, ones[...], (((1,), (0,)), ((), ())),
                         preferred_element_type=jnp.float32)
    inv = jnp.where(n2 <= 1e-24, 1e12, lax.rsqrt(n2))
    out[...] = y * inv


def _mlp(xv, w1, b1, w2, b2, ones):
    vec_spec = pl.BlockSpec((BLK, D), lambda i: (i, 0))
    w_spec = pl.BlockSpec((D, D), lambda i: (0, 0))
    b_spec = pl.BlockSpec((1, D), lambda i: (0, 0))
    return pl.pallas_call(
        _mlp_body,
        grid=(BATCH // BLK,),
        in_specs=[vec_spec, w_spec, b_spec, w_spec, b_spec, w_spec],
        out_specs=vec_spec,
        out_shape=jax.ShapeDtypeStruct((BATCH, D), jnp.float32),
    )(xv, w1, b1.reshape(1, D), w2, b2.reshape(1, D), ones)


def kernel(user_ids, item_ids, user_table, item_table,
           W1u, b1u, W2u, b2u, W1i, b1i, W2i, b2i):
    uidx = user_ids.astype(jnp.int32).reshape(NW, NCH, CHUNK)
    iidx = item_ids.astype(jnp.int32).reshape(NW, NCH, CHUNK)
    gather = _make_gather()
    u_vecs = gather(user_table, uidx)
    v_vecs = gather(item_table, iidx)
    ones = jnp.ones((D, D), jnp.float32)
    u = _mlp(u_vecs, W1u, b1u, W2u, b2u, ones)
    v = _mlp(v_vecs, W1i, b1i, W2i, b2i, ones)
    return (u, v)


# speedup vs baseline: 1.1748x; 1.0254x over previous
import functools

import jax
import jax.numpy as jnp
from jax import lax
from jax.experimental import pallas as pl
from jax.experimental.pallas import tpu as pltpu
from jax.experimental.pallas import tpu_sc as plsc

BATCH = 16384
D = 128
NC = 2
NS = 16
NW = NC * NS
BPW = BATCH // NW
CHUNK = 128
NCH = BPW // CHUNK


def _gather_body(table, idx_hbm, out, idx_v, rows_v, sem):
    wid = lax.axis_index("s") * NC + lax.axis_index("c")
    base = wid * BPW
    pltpu.sync_copy(idx_hbm.at[wid], idx_v)
    cps = [
        pltpu.async_copy(table.at[idx_v.at[c]],
                         rows_v.at[pl.ds(c * CHUNK, CHUNK)], sem)
        for c in range(NCH)
    ]
    for cp in cps:
        cp.wait()
    pltpu.sync_copy(rows_v, out.at[pl.ds(base, BPW)])


@functools.lru_cache(maxsize=1)
def _make_gather():
    mesh = plsc.VectorSubcoreMesh(core_axis_name="c", subcore_axis_name="s")
    return functools.partial(
        pl.kernel,
        mesh=mesh,
        out_type=jax.ShapeDtypeStruct((BATCH, D), jnp.float32),
        scratch_types=[
            pltpu.VMEM((NCH, CHUNK), jnp.int32),
            pltpu.VMEM((BPW, D), jnp.float32),
            pltpu.SemaphoreType.DMA,
        ],
    )(_gather_body)


BLK = 8192


def _mlp_body(xv, w1, b1, w2, b2, ones, out):
    x = xv[...]
    h = lax.dot_general(x, w1[...], (((1,), (1,)), ((), ())),
                        preferred_element_type=jnp.float32)
    h = jnp.maximum(h + b1[...], 0.0)
    y = lax.dot_general(h, w2[...], (((1,), (1,)), ((), ())),
                        preferred_element_type=jnp.float32) + b2[...]
    n2 = lax.dot_general(y * y, ones[...], (((1,), (0,)), ((), ())),
                         preferred_element_type=jnp.float32)
    inv = jnp.where(n2 <= 1e-24, 1e12, lax.rsqrt(n2))
    out[...] = y * inv


def _mlp(xv, w1, b1, w2, b2, ones):
    vec_spec = pl.BlockSpec((BLK, D), lambda i: (i, 0))
    w_spec = pl.BlockSpec((D, D), lambda i: (0, 0))
    b_spec = pl.BlockSpec((1, D), lambda i: (0, 0))
    return pl.pallas_call(
        _mlp_body,
        grid=(BATCH // BLK,),
        in_specs=[vec_spec, w_spec, b_spec, w_spec, b_spec, w_spec],
        out_specs=vec_spec,
        out_shape=jax.ShapeDtypeStruct((BATCH, D), jnp.float32),
    )(xv, w1, b1.reshape(1, D), w2, b2.reshape(1, D), ones)


def kernel(user_ids, item_ids, user_table, item_table,
           W1u, b1u, W2u, b2u, W1i, b1i, W2i, b2i):
    uidx = user_ids.astype(jnp.int32).reshape(NW, NCH, CHUNK)
    iidx = item_ids.astype(jnp.int32).reshape(NW, NCH, CHUNK)
    gather = _make_gather()
    u_vecs = gather(user_table, uidx)
    v_vecs = gather(item_table, iidx)
    ones = jnp.ones((D, D), jnp.float32)
    u = _mlp(u_vecs, W1u, b1u, W2u, b2u, ones)
    v = _mlp(v_vecs, W1i, b1i, W2i, b2i, ones)
    return (u, v)
